# Initial kernel scaffold; baseline (speedup 1.0000x reference)
import jax
import jax.numpy as jnp
from jax.experimental import pallas as pl


def kernel(field_real, field_imag):
    raise NotImplementedError("write your pallas kernel here")


# speedup vs baseline: 1.3931x; 1.3931x over previous
import numpy as np
import jax
import jax.numpy as jnp
from jax.experimental import pallas as pl

N = 16384
T = 1024
TB = 128
THRESHOLD = 0.1


def _vortex_block(real_ref, imag_ref, vortex_ref, winding_ref):
    r = real_ref[...]
    i = imag_ref[...]
    mean_mag = jnp.mean(jnp.sqrt(r * r + i * i), axis=0)

    r0, r1 = r[:-1, :], r[1:, :]
    i0, i1 = i[:-1, :], i[1:, :]
    cross = r0 * i1 - i0 * r1
    up0 = i0 > 0.0
    up1 = i1 > 0.0
    n_pos = jnp.sum(((~up0) & up1 & (cross < 0.0)).astype(jnp.float32), axis=0)
    n_neg = jnp.sum((up0 & (~up1) & (cross > 0.0)).astype(jnp.float32), axis=0)

    theta_first = jnp.arctan2(i[0, :], r[0, :])
    theta_last = jnp.arctan2(i[-1, :], r[-1, :])
    winding = (theta_last - theta_first) * np.float32(0.5 / np.pi) + (n_neg - n_pos)

    is_v = (mean_mag < THRESHOLD) & (jnp.abs(winding) > 0.5)
    vortex_ref[...] = is_v.astype(jnp.int32)[None, :]
    winding_ref[...] = jnp.where(is_v, winding, 0.0)[None, :]


@jax.jit
def kernel(field_real, field_imag):
    out = pl.pallas_call(
        _vortex_block,
        grid=(T // TB,),
        in_specs=[
            pl.BlockSpec((N, TB), lambda t: (0, t)),
            pl.BlockSpec((N, TB), lambda t: (0, t)),
        ],
        out_specs=[
            pl.BlockSpec((1, TB), lambda t: (0, t)),
            pl.BlockSpec((1, TB), lambda t: (0, t)),
        ],
        out_shape=[
            jax.ShapeDtypeStruct((1, T), jnp.int32),
            jax.ShapeDtypeStruct((1, T), jnp.float32),
        ],
    )(field_real, field_imag)
    return (out[0].reshape(T), out[1].reshape(T))
